# baseline (speedup 1.0000x reference)
import functools
import jax
import jax.numpy as jnp
from jax import lax
from jax.experimental import pallas as pl
from jax.experimental.pallas import tpu as pltpu
from jax.experimental.pallas import tpu_sc as plsc

_N = 10000
_E = 160000
_D = 256
_H = 8
_C = 256
_DFF = 512
_B = 32

_NEG = -3.4e38


def _prep_body(x_ref, wgat_ref, atts_ref, attd_ref,
               as_ref, ad_ref, ws_ref, wd_ref, m_ref):
    i = pl.program_id(0)
    cols_s = []
    cols_d = []
    for h in range(_H):
        wblk = wgat_ref[:, h * _C:(h + 1) * _C]
        cols_s.append(jnp.dot(wblk, atts_ref[h, :], preferred_element_type=jnp.float32))
        cols_d.append(jnp.dot(wblk, attd_ref[h, :], preferred_element_type=jnp.float32))
    ws = jnp.stack(cols_s, axis=1)
    wd = jnp.stack(cols_d, axis=1)
    ws_ref[...] = ws
    wd_ref[...] = wd
    a_s = jnp.dot(x_ref[...], ws, preferred_element_type=jnp.float32)
    a_d = jnp.dot(x_ref[...], wd, preferred_element_type=jnp.float32)
    as_ref[...] = jnp.concatenate([a_s, a_s], axis=1)
    ad_ref[...] = jnp.concatenate([a_d, a_d], axis=1)

    @pl.when(i == 0)
    def _():
        m_ref[...] = jnp.full_like(m_ref, _NEG)
    m_ref[...] = jnp.maximum(m_ref[...],
                             jnp.stack([a_s.max(axis=0), a_d.max(axis=0)]))


def _prep(x, W_gat, att_src, att_dst):
    blk = 2000
    grid = (_N // blk,)
    return pl.pallas_call(
        _prep_body,
        grid=grid,
        in_specs=[
            pl.BlockSpec((blk, _D), lambda i: (i, 0)),
            pl.BlockSpec((_D, _H * _C), lambda i: (0, 0)),
            pl.BlockSpec((_H, _C), lambda i: (0, 0)),
            pl.BlockSpec((_H, _C), lambda i: (0, 0)),
        ],
        out_specs=[
            pl.BlockSpec((blk, 16), lambda i: (i, 0)),
            pl.BlockSpec((blk, 16), lambda i: (i, 0)),
            pl.BlockSpec((_D, _H), lambda i: (0, 0)),
            pl.BlockSpec((_D, _H), lambda i: (0, 0)),
            pl.BlockSpec((2, _H), lambda i: (0, 0)),
        ],
        out_shape=[
            jax.ShapeDtypeStruct((_N, 16), jnp.float32),
            jax.ShapeDtypeStruct((_N, 16), jnp.float32),
            jax.ShapeDtypeStruct((_D, _H), jnp.float32),
            jax.ShapeDtypeStruct((_D, _H), jnp.float32),
            jax.ShapeDtypeStruct((2, _H), jnp.float32),
        ],
    )(x, W_gat, att_src, att_dst)


def _gatout_body(u_ref, wflat_ref, bias_ref, ws_ref, wd_ref,
                 out_ref, as_ref, ad_ref, m_ref):
    i = pl.program_id(0)
    out = jnp.dot(u_ref[...], wflat_ref[...], preferred_element_type=jnp.float32)
    out = out * (1.0 / _H) + bias_ref[...]
    out_ref[...] = out
    a_s = jnp.dot(out, ws_ref[...], preferred_element_type=jnp.float32)
    a_d = jnp.dot(out, wd_ref[...], preferred_element_type=jnp.float32)
    as_ref[...] = jnp.concatenate([a_s, a_s], axis=1)
    ad_ref[...] = jnp.concatenate([a_d, a_d], axis=1)

    @pl.when(i == 0)
    def _():
        m_ref[...] = jnp.full_like(m_ref, _NEG)
    m_ref[...] = jnp.maximum(m_ref[...],
                             jnp.stack([a_s.max(axis=0), a_d.max(axis=0)]))


def _gatout(U, Wflat, bias, Ws, Wd):
    blk = 1000
    grid = (_N // blk,)
    return pl.pallas_call(
        _gatout_body,
        grid=grid,
        in_specs=[
            pl.BlockSpec((blk, _H * _D), lambda i: (i, 0)),
            pl.BlockSpec((_H * _D, _C), lambda i: (0, 0)),
            pl.BlockSpec((1, _C), lambda i: (0, 0)),
            pl.BlockSpec((_D, _H), lambda i: (0, 0)),
            pl.BlockSpec((_D, _H), lambda i: (0, 0)),
        ],
        out_specs=[
            pl.BlockSpec((blk, _C), lambda i: (i, 0)),
            pl.BlockSpec((blk, 16), lambda i: (i, 0)),
            pl.BlockSpec((blk, 16), lambda i: (i, 0)),
            pl.BlockSpec((2, _H), lambda i: (0, 0)),
        ],
        out_shape=[
            jax.ShapeDtypeStruct((_N, _C), jnp.float32),
            jax.ShapeDtypeStruct((_N, 16), jnp.float32),
            jax.ShapeDtypeStruct((_N, 16), jnp.float32),
            jax.ShapeDtypeStruct((2, _H), jnp.float32),
        ],
    )(U, Wflat, bias, Ws, Wd)


def _gatout2_body(u_ref, wflat_ref, bias_ref, out_ref):
    out = jnp.dot(u_ref[...], wflat_ref[...], preferred_element_type=jnp.float32)
    out_ref[...] = out * (1.0 / _H) + bias_ref[...]


def _gatout2(U, Wflat, bias):
    blk = 1000
    return pl.pallas_call(
        _gatout2_body,
        grid=(_N // blk,),
        in_specs=[
            pl.BlockSpec((blk, _H * _D), lambda i: (i, 0)),
            pl.BlockSpec((_H * _D, _C), lambda i: (0, 0)),
            pl.BlockSpec((1, _C), lambda i: (0, 0)),
        ],
        out_specs=pl.BlockSpec((blk, _C), lambda i: (i, 0)),
        out_shape=jax.ShapeDtypeStruct((_N, _C), jnp.float32),
    )(U, Wflat, bias)


def _ffn_body(x1_ref, x2_ref, w1_ref, b1_ref, w2_ref, b2_ref,
              ln1g_ref, ln1b_ref, ln2g_ref, ln2b_ref, gw_ref, gb_ref,
              po_ref, gate_ref, gmax_ref):
    i = pl.program_id(0)
    s = x1_ref[...] + x2_ref[...]
    mu = s.mean(axis=-1, keepdims=True)
    var = ((s - mu) ** 2).mean(axis=-1, keepdims=True)
    pi = (s - mu) * lax.rsqrt(var + 1e-5) * ln1g_ref[...] + ln1b_ref[...]
    hdn = jnp.maximum(jnp.dot(pi, w1_ref[...], preferred_element_type=jnp.float32) + b1_ref[...], 0.0)
    ff = jnp.dot(hdn, w2_ref[...], preferred_element_type=jnp.float32) + b2_ref[...]
    t = pi + ff
    mu2 = t.mean(axis=-1, keepdims=True)
    var2 = ((t - mu2) ** 2).mean(axis=-1, keepdims=True)
    po = (t - mu2) * lax.rsqrt(var2 + 1e-5) * ln2g_ref[...] + ln2b_ref[...]
    po_ref[...] = po
    gate = jnp.dot(po, gw_ref[...], preferred_element_type=jnp.float32) + gb_ref[...]
    gate_ref[...] = gate

    @pl.when(i == 0)
    def _():
        gmax_ref[...] = jnp.full_like(gmax_ref, _NEG)
    gmax_ref[...] = jnp.maximum(gmax_ref[...], gate.max())


def _ffn(x1, x2, W1, b1, W2, b2, ln1g, ln1b, ln2g, ln2b, gW, gb):
    blk = 2000
    c0 = lambda i: (0, 0)
    return pl.pallas_call(
        _ffn_body,
        grid=(_N // blk,),
        in_specs=[
            pl.BlockSpec((blk, _C), lambda i: (i, 0)),
            pl.BlockSpec((blk, _C), lambda i: (i, 0)),
            pl.BlockSpec((_C, _DFF), c0),
            pl.BlockSpec((1, _DFF), c0),
            pl.BlockSpec((_DFF, _C), c0),
            pl.BlockSpec((1, _C), c0),
            pl.BlockSpec((1, _C), c0),
            pl.BlockSpec((1, _C), c0),
            pl.BlockSpec((1, _C), c0),
            pl.BlockSpec((1, _C), c0),
            pl.BlockSpec((_C, 1), c0),
            pl.BlockSpec((1, 1), c0),
        ],
        out_specs=[
            pl.BlockSpec((blk, _C), lambda i: (i, 0)),
            pl.BlockSpec((blk, 1), lambda i: (i, 0)),
            pl.BlockSpec((1, 1), c0),
        ],
        out_shape=[
            jax.ShapeDtypeStruct((_N, _C), jnp.float32),
            jax.ShapeDtypeStruct((_N, 1), jnp.float32),
            jax.ShapeDtypeStruct((1, 1), jnp.float32),
        ],
    )(x1, x2, W1, b1, W2, b2, ln1g, ln1b, ln2g, ln2b, gW, gb)


def _pool_body(po_ref, gate_ref, gmax_ref, batch_ref, out_ref, s_ref, den_ref):
    i = pl.program_id(0)
    nsteps = pl.num_programs(0)

    @pl.when(i == 0)
    def _():
        s_ref[...] = jnp.zeros_like(s_ref)
        den_ref[...] = jnp.zeros_like(den_ref)

    ex = jnp.exp(gate_ref[...] - gmax_ref[...])
    bvec = batch_ref[...]
    bid = jax.lax.broadcasted_iota(jnp.int32, (1, _B), 1)
    P = (bvec == bid).astype(jnp.float32)
    Pex = P * ex
    s_ref[...] += lax.dot_general(Pex, po_ref[...], (((0,), (0,)), ((), ())),
                                  preferred_element_type=jnp.float32)
    den_ref[...] += lax.dot_general(P, ex, (((0,), (0,)), ((), ())),
                                    preferred_element_type=jnp.float32)

    @pl.when(i == nsteps - 1)
    def _():
        out_ref[...] = s_ref[...] / (den_ref[...] + 1e-16)


def _pool(po, gate, gmax, batch2d):
    blk = 2000
    c0 = lambda i: (0, 0)
    return pl.pallas_call(
        _pool_body,
        grid=(_N // blk,),
        in_specs=[
            pl.BlockSpec((blk, _C), lambda i: (i, 0)),
            pl.BlockSpec((blk, 1), lambda i: (i, 0)),
            pl.BlockSpec((1, 1), c0),
            pl.BlockSpec((blk, 1), lambda i: (i, 0)),
        ],
        out_specs=pl.BlockSpec((_B, _C), c0),
        out_shape=jax.ShapeDtypeStruct((_B, _C), jnp.float32),
        scratch_shapes=[
            pltpu.VMEM((_B, _C), jnp.float32),
            pltpu.VMEM((_B, 1), jnp.float32),
        ],
    )(po, gate, gmax, batch2d)


_NC, _NS = 2, 16
_NW = _NC * _NS
_NT = 10240
_TRASH = _N
_EP = 172032
_EPW = _EP // _NW
_B1 = 128
_NB1 = _EPW // _B1
_STR1 = _NT // _NS


def _p1_body(src_hbm, dst_hbm, ast_hbm, adt_hbm, mt_hbm,
             ex_hbm, denp_hbm,
             srcb, dstb, asg, adg, exb, mtv, den_sh, sem):
    c = lax.axis_index("c")
    s = lax.axis_index("s")
    wid = s * _NC + c
    pltpu.sync_copy(mt_hbm, mtv)

    zv = jnp.zeros((16,), jnp.float32)

    @pl.loop(0, _B1)
    def _z(e):
        exb[e, :] = zv

    @pl.loop(0, _STR1 // _B1)
    def _zs(j):
        pltpu.sync_copy(exb, den_sh.at[pl.ds(s * _STR1 + j * _B1, _B1)])

    plsc.subcore_barrier()

    @pl.loop(0, _NB1)
    def _batches(b):
        base = wid * _EPW + b * _B1
        pltpu.sync_copy(src_hbm.at[pl.ds(base, _B1)], srcb)
        pltpu.sync_copy(dst_hbm.at[pl.ds(base, _B1)], dstb)
        cp1 = pltpu.async_copy(ast_hbm.at[srcb], asg, sem)
        cp1.wait()
        cp2 = pltpu.async_copy(adt_hbm.at[dstb], adg, sem)
        cp2.wait()
        mt_reg = mtv[...]

        @pl.loop(0, _B1)
        def _edges(e):
            al = asg[e, :] + adg[e, :]
            al = jnp.where(al >= 0.0, al, 0.2 * al)
            exb[e, :] = jnp.exp(al - mt_reg)

        pltpu.sync_copy(exb, den_sh.at[dstb], add=True)
        pltpu.sync_copy(exb, ex_hbm.at[pl.ds(base, _B1)])

    plsc.subcore_barrier()
    pltpu.sync_copy(den_sh.at[pl.ds(s * _STR1, _STR1)],
                    denp_hbm.at[c, pl.ds(s * _STR1, _STR1)])


def _sc_pass1(src_p, dst_p, ast, adt, mt16):
    f = pl.kernel(
        _p1_body,
        out_type=[
            jax.ShapeDtypeStruct((_EP, 16), jnp.float32),
            jax.ShapeDtypeStruct((_NC, _NT, 16), jnp.float32),
        ],
        mesh=plsc.VectorSubcoreMesh(core_axis_name="c", subcore_axis_name="s",
                                    num_cores=_NC, num_subcores=_NS),
        compiler_params=pltpu.CompilerParams(use_tc_tiling_on_sc=False),
        scratch_types=[
            pltpu.VMEM((_B1,), jnp.int32),
            pltpu.VMEM((_B1,), jnp.int32),
            pltpu.VMEM((_B1, 16), jnp.float32),
            pltpu.VMEM((_B1, 16), jnp.float32),
            pltpu.VMEM((_B1, 16), jnp.float32),
            pltpu.VMEM((16,), jnp.float32),
            pltpu.VMEM_SHARED((_NT, 16), jnp.float32),
            pltpu.SemaphoreType.DMA,
        ],
    )
    return f(src_p, dst_p, ast, adt, mt16)


def _dencomb_body(denp_ref, rden_ref):
    rden_ref[...] = 1.0 / (denp_ref[0] + denp_ref[1] + 1e-16)


def _dencomb(denp):
    return pl.pallas_call(
        _dencomb_body,
        in_specs=[pl.BlockSpec((_NC, _NT, 16), lambda: (0, 0, 0))],
        out_specs=pl.BlockSpec((_NT, 16), lambda: (0, 0)),
        out_shape=jax.ShapeDtypeStruct((_NT, 16), jnp.float32),
    )(denp)


_ETOT = _E + _N


def _edge_u_jax(x_in, coef, src_r, dst_r):
    U = jax.ops.segment_sum(coef[:, :, None] * x_in[src_r][:, None, :], dst_r,
                            num_segments=_N)
    return U.reshape(_N, _H * _D)


def _mt16(m):
    M = m[0] + m[1]
    M = jnp.where(M >= 0, M, 0.2 * M)
    return jnp.concatenate([M, M])


def _padtab(t):
    return jnp.concatenate([t, jnp.zeros((_NT - _N, 16), jnp.float32)])


def kernel(x, edge_index, batch, W_gat, att_src, att_dst, bias_gat,
           W1, b1, W2, b2, ln1_g, ln1_b, ln2_g, ln2_b, gate_W, gate_b):
    loops = jnp.arange(_N, dtype=edge_index.dtype)
    npad = _EP - _ETOT
    src = jnp.concatenate([edge_index[0], loops,
                           jnp.zeros((npad,), edge_index.dtype)])
    dst = jnp.concatenate([edge_index[1], loops,
                           jnp.full((npad,), _TRASH, edge_index.dtype)])
    src_r, dst_r = src[:_ETOT], dst[:_ETOT]

    W3 = W_gat.reshape(_D, _H, _C)
    Wflat = W3.transpose(1, 0, 2).reshape(_H * _D, _C)
    bias2 = bias_gat.reshape(1, _C)

    a_s1, a_d1, Ws, Wd, m1 = _prep(x, W_gat, att_src, att_dst)
    ex1, denp1 = _sc_pass1(src, dst, _padtab(a_s1), _padtab(a_d1), _mt16(m1))
    rden1 = _dencomb(denp1)
    coef1 = ex1[:_ETOT, :_H] * rden1[dst_r, :_H]
    U1 = _edge_u_jax(x, coef1, src_r, dst_r)
    x1, a_s2, a_d2, m2 = _gatout(U1, Wflat, bias2, Ws, Wd)

    ex2, denp2 = _sc_pass1(src, dst, _padtab(a_s2), _padtab(a_d2), _mt16(m2))
    rden2 = _dencomb(denp2)
    coef2 = ex2[:_ETOT, :_H] * rden2[dst_r, :_H]
    U2 = _edge_u_jax(x1, coef2, src_r, dst_r)
    x2 = _gatout2(U2, Wflat, bias2)

    po, gate, gmax = _ffn(x1, x2, W1, b1.reshape(1, _DFF), W2, b2.reshape(1, _C),
                          ln1_g.reshape(1, _C), ln1_b.reshape(1, _C),
                          ln2_g.reshape(1, _C), ln2_b.reshape(1, _C),
                          gate_W, gate_b.reshape(1, 1))
    return _pool(po, gate, gmax, batch.reshape(_N, 1))


# speedup vs baseline: 7.5864x; 7.5864x over previous
import functools
import jax
import jax.numpy as jnp
from jax import lax
from jax.experimental import pallas as pl
from jax.experimental.pallas import tpu as pltpu
from jax.experimental.pallas import tpu_sc as plsc

_N = 10000
_E = 160000
_D = 256
_H = 8
_C = 256
_DFF = 512
_B = 32

_NEG = -3.4e38


def _prep_body(x_ref, wgat_ref, atts_ref, attd_ref,
               as_ref, ad_ref, ws_ref, wd_ref, m_ref):
    i = pl.program_id(0)
    cols_s = []
    cols_d = []
    for h in range(_H):
        wblk = wgat_ref[:, h * _C:(h + 1) * _C]
        cols_s.append(jnp.dot(wblk, atts_ref[h, :], preferred_element_type=jnp.float32))
        cols_d.append(jnp.dot(wblk, attd_ref[h, :], preferred_element_type=jnp.float32))
    ws = jnp.stack(cols_s, axis=1)
    wd = jnp.stack(cols_d, axis=1)
    ws_ref[...] = ws
    wd_ref[...] = wd
    a_s = jnp.dot(x_ref[...], ws, preferred_element_type=jnp.float32)
    a_d = jnp.dot(x_ref[...], wd, preferred_element_type=jnp.float32)
    as_ref[...] = jnp.concatenate([a_s, a_s], axis=1)
    ad_ref[...] = jnp.concatenate([a_d, a_d], axis=1)

    @pl.when(i == 0)
    def _():
        m_ref[...] = jnp.full_like(m_ref, _NEG)
    m_ref[...] = jnp.maximum(m_ref[...],
                             jnp.stack([a_s.max(axis=0), a_d.max(axis=0)]))


def _prep(x, W_gat, att_src, att_dst):
    blk = 2000
    grid = (_N // blk,)
    return pl.pallas_call(
        _prep_body,
        grid=grid,
        in_specs=[
            pl.BlockSpec((blk, _D), lambda i: (i, 0)),
            pl.BlockSpec((_D, _H * _C), lambda i: (0, 0)),
            pl.BlockSpec((_H, _C), lambda i: (0, 0)),
            pl.BlockSpec((_H, _C), lambda i: (0, 0)),
        ],
        out_specs=[
            pl.BlockSpec((blk, 16), lambda i: (i, 0)),
            pl.BlockSpec((blk, 16), lambda i: (i, 0)),
            pl.BlockSpec((_D, _H), lambda i: (0, 0)),
            pl.BlockSpec((_D, _H), lambda i: (0, 0)),
            pl.BlockSpec((2, _H), lambda i: (0, 0)),
        ],
        out_shape=[
            jax.ShapeDtypeStruct((_N, 16), jnp.float32),
            jax.ShapeDtypeStruct((_N, 16), jnp.float32),
            jax.ShapeDtypeStruct((_D, _H), jnp.float32),
            jax.ShapeDtypeStruct((_D, _H), jnp.float32),
            jax.ShapeDtypeStruct((2, _H), jnp.float32),
        ],
    )(x, W_gat, att_src, att_dst)


def _gatout_body(u_ref, wflat_ref, bias_ref, ws_ref, wd_ref,
                 out_ref, as_ref, ad_ref, m_ref):
    i = pl.program_id(0)
    out = jnp.dot(u_ref[...], wflat_ref[...], preferred_element_type=jnp.float32)
    out = out * (1.0 / _H) + bias_ref[...]
    out_ref[...] = out
    a_s = jnp.dot(out, ws_ref[...], preferred_element_type=jnp.float32)
    a_d = jnp.dot(out, wd_ref[...], preferred_element_type=jnp.float32)
    as_ref[...] = jnp.concatenate([a_s, a_s], axis=1)
    ad_ref[...] = jnp.concatenate([a_d, a_d], axis=1)

    @pl.when(i == 0)
    def _():
        m_ref[...] = jnp.full_like(m_ref, _NEG)
    m_ref[...] = jnp.maximum(m_ref[...],
                             jnp.stack([a_s.max(axis=0), a_d.max(axis=0)]))


def _gatout(U, Wflat, bias, Ws, Wd):
    blk = 1000
    grid = (_N // blk,)
    return pl.pallas_call(
        _gatout_body,
        grid=grid,
        in_specs=[
            pl.BlockSpec((blk, _H * _D), lambda i: (i, 0)),
            pl.BlockSpec((_H * _D, _C), lambda i: (0, 0)),
            pl.BlockSpec((1, _C), lambda i: (0, 0)),
            pl.BlockSpec((_D, _H), lambda i: (0, 0)),
            pl.BlockSpec((_D, _H), lambda i: (0, 0)),
        ],
        out_specs=[
            pl.BlockSpec((blk, _C), lambda i: (i, 0)),
            pl.BlockSpec((blk, 16), lambda i: (i, 0)),
            pl.BlockSpec((blk, 16), lambda i: (i, 0)),
            pl.BlockSpec((2, _H), lambda i: (0, 0)),
        ],
        out_shape=[
            jax.ShapeDtypeStruct((_N, _C), jnp.float32),
            jax.ShapeDtypeStruct((_N, 16), jnp.float32),
            jax.ShapeDtypeStruct((_N, 16), jnp.float32),
            jax.ShapeDtypeStruct((2, _H), jnp.float32),
        ],
    )(U, Wflat, bias, Ws, Wd)


def _gatout2_body(u_ref, wflat_ref, bias_ref, out_ref):
    out = jnp.dot(u_ref[...], wflat_ref[...], preferred_element_type=jnp.float32)
    out_ref[...] = out * (1.0 / _H) + bias_ref[...]


def _gatout2(U, Wflat, bias):
    blk = 1000
    return pl.pallas_call(
        _gatout2_body,
        grid=(_N // blk,),
        in_specs=[
            pl.BlockSpec((blk, _H * _D), lambda i: (i, 0)),
            pl.BlockSpec((_H * _D, _C), lambda i: (0, 0)),
            pl.BlockSpec((1, _C), lambda i: (0, 0)),
        ],
        out_specs=pl.BlockSpec((blk, _C), lambda i: (i, 0)),
        out_shape=jax.ShapeDtypeStruct((_N, _C), jnp.float32),
    )(U, Wflat, bias)


def _ffn_body(x1_ref, x2_ref, w1_ref, b1_ref, w2_ref, b2_ref,
              ln1g_ref, ln1b_ref, ln2g_ref, ln2b_ref, gw_ref, gb_ref,
              po_ref, gate_ref, gmax_ref):
    i = pl.program_id(0)
    s = x1_ref[...] + x2_ref[...]
    mu = s.mean(axis=-1, keepdims=True)
    var = ((s - mu) ** 2).mean(axis=-1, keepdims=True)
    pi = (s - mu) * lax.rsqrt(var + 1e-5) * ln1g_ref[...] + ln1b_ref[...]
    hdn = jnp.maximum(jnp.dot(pi, w1_ref[...], preferred_element_type=jnp.float32) + b1_ref[...], 0.0)
    ff = jnp.dot(hdn, w2_ref[...], preferred_element_type=jnp.float32) + b2_ref[...]
    t = pi + ff
    mu2 = t.mean(axis=-1, keepdims=True)
    var2 = ((t - mu2) ** 2).mean(axis=-1, keepdims=True)
    po = (t - mu2) * lax.rsqrt(var2 + 1e-5) * ln2g_ref[...] + ln2b_ref[...]
    po_ref[...] = po
    gate = jnp.dot(po, gw_ref[...], preferred_element_type=jnp.float32) + gb_ref[...]
    gate_ref[...] = gate

    @pl.when(i == 0)
    def _():
        gmax_ref[...] = jnp.full_like(gmax_ref, _NEG)
    gmax_ref[...] = jnp.maximum(gmax_ref[...], gate.max())


def _ffn(x1, x2, W1, b1, W2, b2, ln1g, ln1b, ln2g, ln2b, gW, gb):
    blk = 2000
    c0 = lambda i: (0, 0)
    return pl.pallas_call(
        _ffn_body,
        grid=(_N // blk,),
        in_specs=[
            pl.BlockSpec((blk, _C), lambda i: (i, 0)),
            pl.BlockSpec((blk, _C), lambda i: (i, 0)),
            pl.BlockSpec((_C, _DFF), c0),
            pl.BlockSpec((1, _DFF), c0),
            pl.BlockSpec((_DFF, _C), c0),
            pl.BlockSpec((1, _C), c0),
            pl.BlockSpec((1, _C), c0),
            pl.BlockSpec((1, _C), c0),
            pl.BlockSpec((1, _C), c0),
            pl.BlockSpec((1, _C), c0),
            pl.BlockSpec((_C, 1), c0),
            pl.BlockSpec((1, 1), c0),
        ],
        out_specs=[
            pl.BlockSpec((blk, _C), lambda i: (i, 0)),
            pl.BlockSpec((blk, 1), lambda i: (i, 0)),
            pl.BlockSpec((1, 1), c0),
        ],
        out_shape=[
            jax.ShapeDtypeStruct((_N, _C), jnp.float32),
            jax.ShapeDtypeStruct((_N, 1), jnp.float32),
            jax.ShapeDtypeStruct((1, 1), jnp.float32),
        ],
    )(x1, x2, W1, b1, W2, b2, ln1g, ln1b, ln2g, ln2b, gW, gb)


def _pool_body(po_ref, gate_ref, gmax_ref, batch_ref, out_ref, s_ref, den_ref):
    i = pl.program_id(0)
    nsteps = pl.num_programs(0)

    @pl.when(i == 0)
    def _():
        s_ref[...] = jnp.zeros_like(s_ref)
        den_ref[...] = jnp.zeros_like(den_ref)

    ex = jnp.exp(gate_ref[...] - gmax_ref[...])
    bvec = batch_ref[...]
    bid = jax.lax.broadcasted_iota(jnp.int32, (1, _B), 1)
    P = (bvec == bid).astype(jnp.float32)
    Pex = P * ex
    s_ref[...] += lax.dot_general(Pex, po_ref[...], (((0,), (0,)), ((), ())),
                                  preferred_element_type=jnp.float32)
    den_ref[...] += lax.dot_general(P, ex, (((0,), (0,)), ((), ())),
                                    preferred_element_type=jnp.float32)

    @pl.when(i == nsteps - 1)
    def _():
        out_ref[...] = s_ref[...] / (den_ref[...] + 1e-16)


def _pool(po, gate, gmax, batch2d):
    blk = 2000
    c0 = lambda i: (0, 0)
    return pl.pallas_call(
        _pool_body,
        grid=(_N // blk,),
        in_specs=[
            pl.BlockSpec((blk, _C), lambda i: (i, 0)),
            pl.BlockSpec((blk, 1), lambda i: (i, 0)),
            pl.BlockSpec((1, 1), c0),
            pl.BlockSpec((blk, 1), lambda i: (i, 0)),
        ],
        out_specs=pl.BlockSpec((_B, _C), c0),
        out_shape=jax.ShapeDtypeStruct((_B, _C), jnp.float32),
        scratch_shapes=[
            pltpu.VMEM((_B, _C), jnp.float32),
            pltpu.VMEM((_B, 1), jnp.float32),
        ],
    )(po, gate, gmax, batch2d)


_NC, _NS = 2, 16
_NW = _NC * _NS
_NT = 10240
_TRASH = _N
_EP = 172032
_EPW = _EP // _NW
_B1 = 128
_NB1 = _EPW // _B1
_STR1 = _NT // _NS


def _p1_body(src_hbm, dst_hbm, ast_hbm, adt_hbm, mt_hbm,
             ex_hbm, denp_hbm,
             srcb, dstb, asg, adg, exb, mtv, den_sh, sem):
    c = lax.axis_index("c")
    s = lax.axis_index("s")
    wid = s * _NC + c
    pltpu.sync_copy(mt_hbm, mtv)

    zv = jnp.zeros((16,), jnp.float32)

    @pl.loop(0, _B1)
    def _z(e):
        exb[e, :] = zv

    @pl.loop(0, _STR1 // _B1)
    def _zs(j):
        pltpu.sync_copy(exb, den_sh.at[pl.ds(s * _STR1 + j * _B1, _B1)])

    plsc.subcore_barrier()

    @pl.loop(0, _NB1)
    def _batches(b):
        base = wid * _EPW + b * _B1
        pltpu.sync_copy(src_hbm.at[pl.ds(base, _B1)], srcb)
        pltpu.sync_copy(dst_hbm.at[pl.ds(base, _B1)], dstb)
        cp1 = pltpu.async_copy(ast_hbm.at[srcb], asg, sem)
        cp1.wait()
        cp2 = pltpu.async_copy(adt_hbm.at[dstb], adg, sem)
        cp2.wait()
        mt_reg = mtv[...]

        @pl.loop(0, _B1)
        def _edges(e):
            al = asg[e, :] + adg[e, :]
            al = jnp.where(al >= 0.0, al, 0.2 * al)
            exb[e, :] = jnp.exp(al - mt_reg)

        pltpu.sync_copy(exb, den_sh.at[dstb], add=True)
        pltpu.sync_copy(exb, ex_hbm.at[pl.ds(base, _B1)])

    plsc.subcore_barrier()
    pltpu.sync_copy(den_sh.at[pl.ds(s * _STR1, _STR1)],
                    denp_hbm.at[c, pl.ds(s * _STR1, _STR1)])


def _sc_pass1(src_p, dst_p, ast, adt, mt16):
    f = pl.kernel(
        _p1_body,
        out_type=[
            jax.ShapeDtypeStruct((_EP, 16), jnp.float32),
            jax.ShapeDtypeStruct((_NC, _NT, 16), jnp.float32),
        ],
        mesh=plsc.VectorSubcoreMesh(core_axis_name="c", subcore_axis_name="s",
                                    num_cores=_NC, num_subcores=_NS),
        compiler_params=pltpu.CompilerParams(use_tc_tiling_on_sc=False),
        scratch_types=[
            pltpu.VMEM((_B1,), jnp.int32),
            pltpu.VMEM((_B1,), jnp.int32),
            pltpu.VMEM((_B1, 16), jnp.float32),
            pltpu.VMEM((_B1, 16), jnp.float32),
            pltpu.VMEM((_B1, 16), jnp.float32),
            pltpu.VMEM((16,), jnp.float32),
            pltpu.VMEM_SHARED((_NT, 16), jnp.float32),
            pltpu.SemaphoreType.DMA,
        ],
    )
    return f(src_p, dst_p, ast, adt, mt16)


_CH = 512
_NCHPC = 10
_NCH = _NCHPC * _NC
_NU = _NCH * _CH
_EPT = _EP // _NS
_NBF = _EPT // _B1
_B2 = 16
_MB = 2048
_STR2 = _CH // _NS
_LCAP = 120


def _p2_body(src_hbm, dst_hbm, ex_hbm, rden_hbm, x_hbm,
             u_hbm,
             srcb, dstb, msrc, mdst, meid, xg, exg, rdg, rowbuf, idxs, idxr,
             zbuf, idxg1, idxg2, cvb, slab, sem, sem2, sem3):
    c = lax.axis_index("c")
    s = lax.axis_index("s")

    zv = jnp.zeros((16,), jnp.float32)

    @pl.loop(0, 8)
    def _zr(r):
        for k in range(_H * _D // 16):
            zbuf[r, pl.ds(k * 16, 16)] = zv

    @pl.loop(0, _NCHPC)
    def _chunks(ci):
        ch = c * _NCHPC + ci
        lo = ch * _CH
        hi = lo + _CH

        @pl.loop(0, _STR2 // 8)
        def _zs(j):
            pltpu.sync_copy(zbuf, slab.at[pl.ds(s * _STR2 + j * 8, 8)])

        plsc.subcore_barrier()

        lanebase = lax.iota(jnp.int32, 16) * _LCAP

        def _fbatch(b, cntv):
            ebase = s * _EPT + b * _B1
            pltpu.sync_copy(src_hbm.at[pl.ds(ebase, _B1)], srcb)
            pltpu.sync_copy(dst_hbm.at[pl.ds(ebase, _B1)], dstb)
            for j in range(_B1 // 16):
                dv = dstb[pl.ds(j * 16, 16)]
                sv = srcb[pl.ds(j * 16, 16)]
                ev = lax.iota(jnp.int32, 16) + (ebase + j * 16)
                m = (dv >= lo) & (dv < hi)
                mi = m.astype(jnp.int32)
                pos = jnp.where(m, lanebase + cntv,
                                _MB - 16 + lax.iota(jnp.int32, 16))
                plsc.store_scatter(mdst, [pos], dv)
                plsc.store_scatter(msrc, [pos], sv)
                plsc.store_scatter(meid, [pos], ev)
                cntv = cntv + mi
            return cntv

        cntv = lax.fori_loop(0, _NBF, _fbatch,
                             jnp.zeros((16,), jnp.int32))

        padv = jnp.full((16,), 1 << 20, jnp.int32)
        zi = jnp.zeros((16,), jnp.int32)
        for k in range(16):
            ppos = lanebase + cntv + k
            plsc.store_scatter(mdst, [ppos], padv)
            plsc.store_scatter(msrc, [ppos], zi)
            plsc.store_scatter(meid, [ppos], zi)
        cvb[...] = cntv

        for L in range(16):
            cl = cvb[...][L]
            nbL = (cl + 15) // 16

            @pl.loop(0, nbL)
            def _accum(b):
                t = L * _LCAP + b * 16
                gd = mdst[pl.ds(t, 16)]
                gs = jnp.clip(msrc[pl.ds(t, 16)], 0, _N - 1)
                ge = jnp.clip(meid[pl.ds(t, 16)], 0, _EP - 1)
                ils = jnp.clip(gd - lo, 0, _CH)
                ir = jnp.clip(gd, 0, _NT - 1)
                cp1 = pltpu.async_copy(x_hbm.at[gs], xg, sem)
                cp1.wait()
                cp2 = pltpu.async_copy(ex_hbm.at[ge], exg, sem2)
                cp2.wait()
                cp3 = pltpu.async_copy(rden_hbm.at[ir], rdg, sem3)
                cp3.wait()

                @pl.loop(0, _B2)
                def _edge(e):
                    coef = exg[e, :] * rdg[e, :]
                    for h in range(_H):
                        cb = lax.broadcast_in_dim(coef[h], (16,), ())
                        for k in range(_D // 16):
                            rowbuf[e, pl.ds(h * _D + k * 16, 16)] = (
                                cb * xg[e, pl.ds(k * 16, 16)])

                pltpu.sync_copy(rowbuf, slab.at[ils], add=True)

        plsc.subcore_barrier()
        pltpu.sync_copy(slab.at[pl.ds(s * _STR2, _STR2)],
                        u_hbm.at[pl.ds(lo + s * _STR2, _STR2)])
        plsc.subcore_barrier()


def _sc_pass2(src_p, dst_p, ex, rden, x_in):
    f = pl.kernel(
        _p2_body,
        out_type=jax.ShapeDtypeStruct((_NU, _H * _D), jnp.float32),
        mesh=plsc.VectorSubcoreMesh(core_axis_name="c", subcore_axis_name="s",
                                    num_cores=_NC, num_subcores=_NS),
        compiler_params=pltpu.CompilerParams(use_tc_tiling_on_sc=False,
                                             needs_layout_passes=False),
        scratch_types=[
            pltpu.VMEM((_B1,), jnp.int32),
            pltpu.VMEM((_B1,), jnp.int32),
            pltpu.VMEM((_MB,), jnp.int32),
            pltpu.VMEM((_MB,), jnp.int32),
            pltpu.VMEM((_MB,), jnp.int32),
            pltpu.VMEM((_B2, _D), jnp.float32),
            pltpu.VMEM((_B2, 16), jnp.float32),
            pltpu.VMEM((_B2, 16), jnp.float32),
            pltpu.VMEM((_B2, _H * _D), jnp.float32),
            pltpu.VMEM((_B2,), jnp.int32),
            pltpu.VMEM((_B2,), jnp.int32),
            pltpu.VMEM((8, _H * _D), jnp.float32),
            pltpu.VMEM((_B2,), jnp.int32),
            pltpu.VMEM((_B2,), jnp.int32),
            pltpu.VMEM((16,), jnp.int32),
            pltpu.VMEM_SHARED((_CH + 8, _H * _D), jnp.float32),
            pltpu.SemaphoreType.DMA,
            pltpu.SemaphoreType.DMA,
            pltpu.SemaphoreType.DMA,
        ],
    )
    return f(src_p, dst_p, ex, rden, x_in)


def _dencomb_body(denp_ref, rden_ref):
    rden_ref[...] = 1.0 / (denp_ref[0] + denp_ref[1] + 1e-16)


def _dencomb(denp):
    return pl.pallas_call(
        _dencomb_body,
        in_specs=[pl.BlockSpec((_NC, _NT, 16), lambda: (0, 0, 0))],
        out_specs=pl.BlockSpec((_NT, 16), lambda: (0, 0)),
        out_shape=jax.ShapeDtypeStruct((_NT, 16), jnp.float32),
    )(denp)


_ETOT = _E + _N


def _edge_u_jax(x_in, coef, src_r, dst_r):
    U = jax.ops.segment_sum(coef[:, :, None] * x_in[src_r][:, None, :], dst_r,
                            num_segments=_N)
    return U.reshape(_N, _H * _D)


def _mt16(m):
    M = m[0] + m[1]
    M = jnp.where(M >= 0, M, 0.2 * M)
    return jnp.concatenate([M, M])


def _padtab(t):
    return jnp.concatenate([t, jnp.zeros((_NT - _N, 16), jnp.float32)])


def kernel(x, edge_index, batch, W_gat, att_src, att_dst, bias_gat,
           W1, b1, W2, b2, ln1_g, ln1_b, ln2_g, ln2_b, gate_W, gate_b):
    loops = jnp.arange(_N, dtype=edge_index.dtype)
    npad = _EP - _ETOT
    src = jnp.concatenate([edge_index[0], loops,
                           jnp.zeros((npad,), edge_index.dtype)])
    dst = jnp.concatenate([edge_index[1], loops,
                           jnp.full((npad,), _TRASH, edge_index.dtype)])
    src_r, dst_r = src[:_ETOT], dst[:_ETOT]

    W3 = W_gat.reshape(_D, _H, _C)
    Wflat = W3.transpose(1, 0, 2).reshape(_H * _D, _C)
    bias2 = bias_gat.reshape(1, _C)

    a_s1, a_d1, Ws, Wd, m1 = _prep(x, W_gat, att_src, att_dst)
    ex1, denp1 = _sc_pass1(src, dst, _padtab(a_s1), _padtab(a_d1), _mt16(m1))
    rden1 = _dencomb(denp1)
    U1 = _sc_pass2(src, dst, ex1, rden1, x)
    x1, a_s2, a_d2, m2 = _gatout(U1, Wflat, bias2, Ws, Wd)

    ex2, denp2 = _sc_pass1(src, dst, _padtab(a_s2), _padtab(a_d2), _mt16(m2))
    rden2 = _dencomb(denp2)
    U2 = _sc_pass2(src, dst, ex2, rden2, x1)
    x2 = _gatout2(U2, Wflat, bias2)

    po, gate, gmax = _ffn(x1, x2, W1, b1.reshape(1, _DFF), W2, b2.reshape(1, _C),
                          ln1_g.reshape(1, _C), ln1_b.reshape(1, _C),
                          ln2_g.reshape(1, _C), ln2_b.reshape(1, _C),
                          gate_W, gate_b.reshape(1, 1))
    return _pool(po, gate, gmax, batch.reshape(_N, 1))


# speedup vs baseline: 8.2064x; 1.0817x over previous
import functools
import jax
import jax.numpy as jnp
from jax import lax
from jax.experimental import pallas as pl
from jax.experimental.pallas import tpu as pltpu
from jax.experimental.pallas import tpu_sc as plsc

_N = 10000
_E = 160000
_D = 256
_H = 8
_C = 256
_DFF = 512
_B = 32

_NEG = -3.4e38


def _prep_body(x_ref, wgat_ref, atts_ref, attd_ref,
               as_ref, ad_ref, ws_ref, wd_ref, m_ref):
    i = pl.program_id(0)
    cols_s = []
    cols_d = []
    for h in range(_H):
        wblk = wgat_ref[:, h * _C:(h + 1) * _C]
        cols_s.append(jnp.dot(wblk, atts_ref[h, :], preferred_element_type=jnp.float32))
        cols_d.append(jnp.dot(wblk, attd_ref[h, :], preferred_element_type=jnp.float32))
    ws = jnp.stack(cols_s, axis=1)
    wd = jnp.stack(cols_d, axis=1)
    ws_ref[...] = ws
    wd_ref[...] = wd
    a_s = jnp.dot(x_ref[...], ws, preferred_element_type=jnp.float32)
    a_d = jnp.dot(x_ref[...], wd, preferred_element_type=jnp.float32)
    as_ref[...] = jnp.concatenate([a_s, a_s], axis=1)
    ad_ref[...] = jnp.concatenate([a_d, a_d], axis=1)

    @pl.when(i == 0)
    def _():
        m_ref[...] = jnp.full_like(m_ref, _NEG)
    m_ref[...] = jnp.maximum(m_ref[...],
                             jnp.stack([a_s.max(axis=0), a_d.max(axis=0)]))


def _prep(x, W_gat, att_src, att_dst):
    blk = 2000
    grid = (_N // blk,)
    return pl.pallas_call(
        _prep_body,
        grid=grid,
        in_specs=[
            pl.BlockSpec((blk, _D), lambda i: (i, 0)),
            pl.BlockSpec((_D, _H * _C), lambda i: (0, 0)),
            pl.BlockSpec((_H, _C), lambda i: (0, 0)),
            pl.BlockSpec((_H, _C), lambda i: (0, 0)),
        ],
        out_specs=[
            pl.BlockSpec((blk, 16), lambda i: (i, 0)),
            pl.BlockSpec((blk, 16), lambda i: (i, 0)),
            pl.BlockSpec((_D, _H), lambda i: (0, 0)),
            pl.BlockSpec((_D, _H), lambda i: (0, 0)),
            pl.BlockSpec((2, _H), lambda i: (0, 0)),
        ],
        out_shape=[
            jax.ShapeDtypeStruct((_N, 16), jnp.float32),
            jax.ShapeDtypeStruct((_N, 16), jnp.float32),
            jax.ShapeDtypeStruct((_D, _H), jnp.float32),
            jax.ShapeDtypeStruct((_D, _H), jnp.float32),
            jax.ShapeDtypeStruct((2, _H), jnp.float32),
        ],
    )(x, W_gat, att_src, att_dst)


def _gatout_body(u_ref, wflat_ref, bias_ref, ws_ref, wd_ref,
                 out_ref, as_ref, ad_ref, m_ref):
    i = pl.program_id(0)
    out = jnp.dot(u_ref[...], wflat_ref[...], preferred_element_type=jnp.float32)
    out = out * (1.0 / _H) + bias_ref[...]
    out_ref[...] = out
    a_s = jnp.dot(out, ws_ref[...], preferred_element_type=jnp.float32)
    a_d = jnp.dot(out, wd_ref[...], preferred_element_type=jnp.float32)
    as_ref[...] = jnp.concatenate([a_s, a_s], axis=1)
    ad_ref[...] = jnp.concatenate([a_d, a_d], axis=1)

    @pl.when(i == 0)
    def _():
        m_ref[...] = jnp.full_like(m_ref, _NEG)
    m_ref[...] = jnp.maximum(m_ref[...],
                             jnp.stack([a_s.max(axis=0), a_d.max(axis=0)]))


def _gatout(U, Wflat, bias, Ws, Wd):
    blk = 1000
    grid = (_N // blk,)
    return pl.pallas_call(
        _gatout_body,
        grid=grid,
        in_specs=[
            pl.BlockSpec((blk, _H * _D), lambda i: (i, 0)),
            pl.BlockSpec((_H * _D, _C), lambda i: (0, 0)),
            pl.BlockSpec((1, _C), lambda i: (0, 0)),
            pl.BlockSpec((_D, _H), lambda i: (0, 0)),
            pl.BlockSpec((_D, _H), lambda i: (0, 0)),
        ],
        out_specs=[
            pl.BlockSpec((blk, _C), lambda i: (i, 0)),
            pl.BlockSpec((blk, 16), lambda i: (i, 0)),
            pl.BlockSpec((blk, 16), lambda i: (i, 0)),
            pl.BlockSpec((2, _H), lambda i: (0, 0)),
        ],
        out_shape=[
            jax.ShapeDtypeStruct((_N, _C), jnp.float32),
            jax.ShapeDtypeStruct((_N, 16), jnp.float32),
            jax.ShapeDtypeStruct((_N, 16), jnp.float32),
            jax.ShapeDtypeStruct((2, _H), jnp.float32),
        ],
    )(U, Wflat, bias, Ws, Wd)


def _gatout2_body(u_ref, wflat_ref, bias_ref, out_ref):
    out = jnp.dot(u_ref[...], wflat_ref[...], preferred_element_type=jnp.float32)
    out_ref[...] = out * (1.0 / _H) + bias_ref[...]


def _gatout2(U, Wflat, bias):
    blk = 1000
    return pl.pallas_call(
        _gatout2_body,
        grid=(_N // blk,),
        in_specs=[
            pl.BlockSpec((blk, _H * _D), lambda i: (i, 0)),
            pl.BlockSpec((_H * _D, _C), lambda i: (0, 0)),
            pl.BlockSpec((1, _C), lambda i: (0, 0)),
        ],
        out_specs=pl.BlockSpec((blk, _C), lambda i: (i, 0)),
        out_shape=jax.ShapeDtypeStruct((_N, _C), jnp.float32),
    )(U, Wflat, bias)


def _ffn_body(x1_ref, x2_ref, w1_ref, b1_ref, w2_ref, b2_ref,
              ln1g_ref, ln1b_ref, ln2g_ref, ln2b_ref, gw_ref, gb_ref,
              po_ref, gate_ref, gmax_ref):
    i = pl.program_id(0)
    s = x1_ref[...] + x2_ref[...]
    mu = s.mean(axis=-1, keepdims=True)
    var = ((s - mu) ** 2).mean(axis=-1, keepdims=True)
    pi = (s - mu) * lax.rsqrt(var + 1e-5) * ln1g_ref[...] + ln1b_ref[...]
    hdn = jnp.maximum(jnp.dot(pi, w1_ref[...], preferred_element_type=jnp.float32) + b1_ref[...], 0.0)
    ff = jnp.dot(hdn, w2_ref[...], preferred_element_type=jnp.float32) + b2_ref[...]
    t = pi + ff
    mu2 = t.mean(axis=-1, keepdims=True)
    var2 = ((t - mu2) ** 2).mean(axis=-1, keepdims=True)
    po = (t - mu2) * lax.rsqrt(var2 + 1e-5) * ln2g_ref[...] + ln2b_ref[...]
    po_ref[...] = po
    gate = jnp.dot(po, gw_ref[...], preferred_element_type=jnp.float32) + gb_ref[...]
    gate_ref[...] = gate

    @pl.when(i == 0)
    def _():
        gmax_ref[...] = jnp.full_like(gmax_ref, _NEG)
    gmax_ref[...] = jnp.maximum(gmax_ref[...], gate.max())


def _ffn(x1, x2, W1, b1, W2, b2, ln1g, ln1b, ln2g, ln2b, gW, gb):
    blk = 2000
    c0 = lambda i: (0, 0)
    return pl.pallas_call(
        _ffn_body,
        grid=(_N // blk,),
        in_specs=[
            pl.BlockSpec((blk, _C), lambda i: (i, 0)),
            pl.BlockSpec((blk, _C), lambda i: (i, 0)),
            pl.BlockSpec((_C, _DFF), c0),
            pl.BlockSpec((1, _DFF), c0),
            pl.BlockSpec((_DFF, _C), c0),
            pl.BlockSpec((1, _C), c0),
            pl.BlockSpec((1, _C), c0),
            pl.BlockSpec((1, _C), c0),
            pl.BlockSpec((1, _C), c0),
            pl.BlockSpec((1, _C), c0),
            pl.BlockSpec((_C, 1), c0),
            pl.BlockSpec((1, 1), c0),
        ],
        out_specs=[
            pl.BlockSpec((blk, _C), lambda i: (i, 0)),
            pl.BlockSpec((blk, 1), lambda i: (i, 0)),
            pl.BlockSpec((1, 1), c0),
        ],
        out_shape=[
            jax.ShapeDtypeStruct((_N, _C), jnp.float32),
            jax.ShapeDtypeStruct((_N, 1), jnp.float32),
            jax.ShapeDtypeStruct((1, 1), jnp.float32),
        ],
    )(x1, x2, W1, b1, W2, b2, ln1g, ln1b, ln2g, ln2b, gW, gb)


def _pool_body(po_ref, gate_ref, gmax_ref, batch_ref, out_ref, s_ref, den_ref):
    i = pl.program_id(0)
    nsteps = pl.num_programs(0)

    @pl.when(i == 0)
    def _():
        s_ref[...] = jnp.zeros_like(s_ref)
        den_ref[...] = jnp.zeros_like(den_ref)

    ex = jnp.exp(gate_ref[...] - gmax_ref[...])
    bvec = batch_ref[...]
    bid = jax.lax.broadcasted_iota(jnp.int32, (1, _B), 1)
    P = (bvec == bid).astype(jnp.float32)
    Pex = P * ex
    s_ref[...] += lax.dot_general(Pex, po_ref[...], (((0,), (0,)), ((), ())),
                                  preferred_element_type=jnp.float32)
    den_ref[...] += lax.dot_general(P, ex, (((0,), (0,)), ((), ())),
                                    preferred_element_type=jnp.float32)

    @pl.when(i == nsteps - 1)
    def _():
        out_ref[...] = s_ref[...] / (den_ref[...] + 1e-16)


def _pool(po, gate, gmax, batch2d):
    blk = 2000
    c0 = lambda i: (0, 0)
    return pl.pallas_call(
        _pool_body,
        grid=(_N // blk,),
        in_specs=[
            pl.BlockSpec((blk, _C), lambda i: (i, 0)),
            pl.BlockSpec((blk, 1), lambda i: (i, 0)),
            pl.BlockSpec((1, 1), c0),
            pl.BlockSpec((blk, 1), lambda i: (i, 0)),
        ],
        out_specs=pl.BlockSpec((_B, _C), c0),
        out_shape=jax.ShapeDtypeStruct((_B, _C), jnp.float32),
        scratch_shapes=[
            pltpu.VMEM((_B, _C), jnp.float32),
            pltpu.VMEM((_B, 1), jnp.float32),
        ],
    )(po, gate, gmax, batch2d)


_NC, _NS = 2, 16
_NW = _NC * _NS
_NT = 10240
_TRASH = _N
_EP = 172032
_EPW = _EP // _NW
_B1 = 128
_NB1 = _EPW // _B1
_STR1 = _NT // _NS


def _p1_body(src_hbm, dst_hbm, ast_hbm, adt_hbm, mt_hbm,
             ex_hbm, denp_hbm,
             srcb, dstb, asg, adg, exb, mtv, den_sh, sem):
    c = lax.axis_index("c")
    s = lax.axis_index("s")
    wid = s * _NC + c
    pltpu.sync_copy(mt_hbm, mtv)

    zv = jnp.zeros((16,), jnp.float32)

    @pl.loop(0, _B1)
    def _z(e):
        exb[e, :] = zv

    @pl.loop(0, _STR1 // _B1)
    def _zs(j):
        pltpu.sync_copy(exb, den_sh.at[pl.ds(s * _STR1 + j * _B1, _B1)])

    plsc.subcore_barrier()

    @pl.loop(0, _NB1)
    def _batches(b):
        base = wid * _EPW + b * _B1
        pltpu.sync_copy(src_hbm.at[pl.ds(base, _B1)], srcb)
        pltpu.sync_copy(dst_hbm.at[pl.ds(base, _B1)], dstb)
        cp1 = pltpu.async_copy(ast_hbm.at[srcb], asg, sem)
        cp1.wait()
        cp2 = pltpu.async_copy(adt_hbm.at[dstb], adg, sem)
        cp2.wait()
        mt_reg = mtv[...]

        @pl.loop(0, _B1)
        def _edges(e):
            al = asg[e, :] + adg[e, :]
            al = jnp.where(al >= 0.0, al, 0.2 * al)
            exb[e, :] = jnp.exp(al - mt_reg)

        pltpu.sync_copy(exb, den_sh.at[dstb], add=True)
        pltpu.sync_copy(exb, ex_hbm.at[pl.ds(base, _B1)])

    plsc.subcore_barrier()
    pltpu.sync_copy(den_sh.at[pl.ds(s * _STR1, _STR1)],
                    denp_hbm.at[c, pl.ds(s * _STR1, _STR1)])


def _sc_pass1(src_p, dst_p, ast, adt, mt16):
    f = pl.kernel(
        _p1_body,
        out_type=[
            jax.ShapeDtypeStruct((_EP, 16), jnp.float32),
            jax.ShapeDtypeStruct((_NC, _NT, 16), jnp.float32),
        ],
        mesh=plsc.VectorSubcoreMesh(core_axis_name="c", subcore_axis_name="s",
                                    num_cores=_NC, num_subcores=_NS),
        compiler_params=pltpu.CompilerParams(use_tc_tiling_on_sc=False),
        scratch_types=[
            pltpu.VMEM((_B1,), jnp.int32),
            pltpu.VMEM((_B1,), jnp.int32),
            pltpu.VMEM((_B1, 16), jnp.float32),
            pltpu.VMEM((_B1, 16), jnp.float32),
            pltpu.VMEM((_B1, 16), jnp.float32),
            pltpu.VMEM((16,), jnp.float32),
            pltpu.VMEM_SHARED((_NT, 16), jnp.float32),
            pltpu.SemaphoreType.DMA,
        ],
    )
    return f(src_p, dst_p, ast, adt, mt16)


_CH = 512
_NCHPC = 10
_NCH = _NCHPC * _NC
_NU = _NCH * _CH
_EPT = _EP // _NS
_NBF = _EPT // _B1
_B2 = 16
_MB = 2048
_STR2 = _CH // _NS
_LCAP = 120


def _p2_body(src_hbm, dst_hbm, ex_hbm, rden_hbm, x_hbm,
             u_hbm,
             srcb, dstb, msrc, mdst, meid, xg, exg, rdg, rowbuf, idxs, idxr,
             zbuf, idxg1, idxg2, cvb, slab, sem, sem2, sem3):
    c = lax.axis_index("c")
    s = lax.axis_index("s")

    zv = jnp.zeros((16,), jnp.float32)

    @pl.loop(0, 8)
    def _zr(r):
        for k in range(_H * _D // 16):
            zbuf[r, pl.ds(k * 16, 16)] = zv

    @pl.loop(0, _NCHPC)
    def _chunks(ci):
        ch = c * _NCHPC + ci
        lo = ch * _CH
        hi = lo + _CH

        @pl.loop(0, _STR2 // 8)
        def _zs(j):
            pltpu.sync_copy(zbuf, slab.at[pl.ds(s * _STR2 + j * 8, 8)])

        plsc.subcore_barrier()

        lanebase = lax.iota(jnp.int32, 16) * _LCAP

        def _fbatch(b, cntv):
            ebase = s * _EPT + b * _B1
            pltpu.sync_copy(src_hbm.at[pl.ds(ebase, _B1)], srcb)
            pltpu.sync_copy(dst_hbm.at[pl.ds(ebase, _B1)], dstb)
            for j in range(_B1 // 16):
                dv = dstb[pl.ds(j * 16, 16)]
                sv = srcb[pl.ds(j * 16, 16)]
                ev = lax.iota(jnp.int32, 16) + (ebase + j * 16)
                m = (dv >= lo) & (dv < hi) & (ev < _ETOT)
                mi = m.astype(jnp.int32)
                pos = jnp.where(m, lanebase + cntv,
                                _MB - 16 + lax.iota(jnp.int32, 16))
                plsc.store_scatter(mdst, [pos], dv)
                plsc.store_scatter(msrc, [pos], sv)
                plsc.store_scatter(meid, [pos], ev)
                cntv = cntv + mi
            return cntv

        cntv = lax.fori_loop(0, _NBF, _fbatch,
                             jnp.zeros((16,), jnp.int32))

        padv = jnp.full((16,), 1 << 20, jnp.int32)
        zi = jnp.zeros((16,), jnp.int32)
        for k in range(16):
            ppos = lanebase + cntv + k
            plsc.store_scatter(mdst, [ppos], padv)
            plsc.store_scatter(msrc, [ppos], zi)
            plsc.store_scatter(meid, [ppos], zi)
        cvb[...] = cntv

        for L in range(16):
            cl = cvb[...][L]
            nbL = (cl + 15) // 16

            @pl.loop(0, nbL)
            def _accum(b):
                t = L * _LCAP + b * 16
                gd = mdst[pl.ds(t, 16)]
                gs = jnp.clip(msrc[pl.ds(t, 16)], 0, _N - 1)
                ge = jnp.clip(meid[pl.ds(t, 16)], 0, _EP - 1)
                ils = jnp.clip(gd - lo, 0, _CH)
                ir = jnp.clip(gd, 0, _NT - 1)
                cp1 = pltpu.async_copy(x_hbm.at[gs], xg, sem)
                cp1.wait()
                cp2 = pltpu.async_copy(ex_hbm.at[ge], exg, sem2)
                cp2.wait()
                cp3 = pltpu.async_copy(rden_hbm.at[ir], rdg, sem3)
                cp3.wait()

                @pl.loop(0, _B2)
                def _edge(e):
                    coef = exg[e, :] * rdg[e, :]
                    for h in range(_H):
                        cb = lax.broadcast_in_dim(coef[h], (16,), ())
                        for k in range(_D // 16):
                            rowbuf[e, pl.ds(h * _D + k * 16, 16)] = (
                                cb * xg[e, pl.ds(k * 16, 16)])

                pltpu.sync_copy(rowbuf, slab.at[ils], add=True)

        plsc.subcore_barrier()
        pltpu.sync_copy(slab.at[pl.ds(s * _STR2, _STR2)],
                        u_hbm.at[pl.ds(lo + s * _STR2, _STR2)])
        plsc.subcore_barrier()


def _sc_pass2(src_p, dst_p, ex, rden, x_in):
    f = pl.kernel(
        _p2_body,
        out_type=jax.ShapeDtypeStruct((_NU, _H * _D), jnp.float32),
        mesh=plsc.VectorSubcoreMesh(core_axis_name="c", subcore_axis_name="s",
                                    num_cores=_NC, num_subcores=_NS),
        compiler_params=pltpu.CompilerParams(use_tc_tiling_on_sc=False,
                                             needs_layout_passes=False),
        scratch_types=[
            pltpu.VMEM((_B1,), jnp.int32),
            pltpu.VMEM((_B1,), jnp.int32),
            pltpu.VMEM((_MB,), jnp.int32),
            pltpu.VMEM((_MB,), jnp.int32),
            pltpu.VMEM((_MB,), jnp.int32),
            pltpu.VMEM((_B2, _D), jnp.float32),
            pltpu.VMEM((_B2, 16), jnp.float32),
            pltpu.VMEM((_B2, 16), jnp.float32),
            pltpu.VMEM((_B2, _H * _D), jnp.float32),
            pltpu.VMEM((_B2,), jnp.int32),
            pltpu.VMEM((_B2,), jnp.int32),
            pltpu.VMEM((8, _H * _D), jnp.float32),
            pltpu.VMEM((_B2,), jnp.int32),
            pltpu.VMEM((_B2,), jnp.int32),
            pltpu.VMEM((16,), jnp.int32),
            pltpu.VMEM_SHARED((_CH + 8, _H * _D), jnp.float32),
            pltpu.SemaphoreType.DMA,
            pltpu.SemaphoreType.DMA,
            pltpu.SemaphoreType.DMA,
        ],
    )
    return f(src_p, dst_p, ex, rden, x_in)


def _dencomb_body(denp_ref, rden_ref):
    rden_ref[...] = 1.0 / (denp_ref[0] + denp_ref[1] + 1e-16)


def _dencomb(denp):
    return pl.pallas_call(
        _dencomb_body,
        in_specs=[pl.BlockSpec((_NC, _NT, 16), lambda: (0, 0, 0))],
        out_specs=pl.BlockSpec((_NT, 16), lambda: (0, 0)),
        out_shape=jax.ShapeDtypeStruct((_NT, 16), jnp.float32),
    )(denp)


_ETOT = _E + _N


def _edge_u_jax(x_in, coef, src_r, dst_r):
    U = jax.ops.segment_sum(coef[:, :, None] * x_in[src_r][:, None, :], dst_r,
                            num_segments=_N)
    return U.reshape(_N, _H * _D)


def _mt16(m):
    M = m[0] + m[1]
    M = jnp.where(M >= 0, M, 0.2 * M)
    return jnp.concatenate([M, M])


def _padtab(t):
    return jnp.concatenate([t, jnp.zeros((_NT - _N, 16), jnp.float32)])


def kernel(x, edge_index, batch, W_gat, att_src, att_dst, bias_gat,
           W1, b1, W2, b2, ln1_g, ln1_b, ln2_g, ln2_b, gate_W, gate_b):
    loops = jnp.arange(_N, dtype=edge_index.dtype)
    npad = _EP - _ETOT
    src = jnp.concatenate([edge_index[0], loops,
                           jnp.zeros((npad,), edge_index.dtype)])
    dst = jnp.concatenate([edge_index[1], loops,
                           jnp.full((npad,), _TRASH, edge_index.dtype)])
    src_r, dst_r = src[:_ETOT], dst[:_ETOT]

    W3 = W_gat.reshape(_D, _H, _C)
    Wflat = W3.transpose(1, 0, 2).reshape(_H * _D, _C)
    bias2 = bias_gat.reshape(1, _C)

    a_s1, a_d1, Ws, Wd, m1 = _prep(x, W_gat, att_src, att_dst)
    ex1, denp1 = _sc_pass1(src, dst, _padtab(a_s1), _padtab(a_d1), _mt16(m1))
    rden1 = _dencomb(denp1)
    U1 = _sc_pass2(src, dst, ex1, rden1, x)
    x1, a_s2, a_d2, m2 = _gatout(U1, Wflat, bias2, Ws, Wd)

    ex2, denp2 = _sc_pass1(src, dst, _padtab(a_s2), _padtab(a_d2), _mt16(m2))
    rden2 = _dencomb(denp2)
    U2 = _sc_pass2(src, dst, ex2, rden2, x1)
    x2 = _gatout2(U2, Wflat, bias2)

    po, gate, gmax = _ffn(x1, x2, W1, b1.reshape(1, _DFF), W2, b2.reshape(1, _C),
                          ln1_g.reshape(1, _C), ln1_b.reshape(1, _C),
                          ln2_g.reshape(1, _C), ln2_b.reshape(1, _C),
                          gate_W, gate_b.reshape(1, 1))
    return _pool(po, gate, gmax, batch.reshape(_N, 1))
